# Initial kernel scaffold; baseline (speedup 1.0000x reference)
import jax
import jax.numpy as jnp
from jax.experimental import pallas as pl


def kernel(queries, keys):
    raise NotImplementedError("write your pallas kernel here")


# speedup vs baseline: 65.5756x; 65.5756x over previous
import functools

import jax
import jax.numpy as jnp
from jax import lax
from jax.experimental import pallas as pl
from jax.experimental.pallas import tpu as pltpu

_INT_MAX = jnp.iinfo(jnp.int32).max


def _pick_tile(n):
    for t in (25000, 20000, 10000, 8000, 5000, 4000, 2000, 1000, 512, 256, 128, 64, 32, 16, 8):
        if n % t == 0:
            return t
    return n


def _scan_body(q_ref, k_ref, td_ref, idx_ref, rt_ref, *, tile):
    i = pl.program_id(0)

    @pl.when(i == 0)
    def _init():
        td_ref[...] = jnp.full(td_ref.shape, jnp.inf, jnp.float32)
        idx_ref[...] = jnp.zeros(idx_ref.shape, jnp.int32)
        rt_ref[...] = jnp.zeros(rt_ref.shape, jnp.float32)

    q = q_ref[...]
    kt = k_ref[...]
    qq = jnp.sum(q * q, axis=1, keepdims=True)
    ksq = lax.dot_general(jnp.ones((1, q.shape[1]), jnp.float32), kt * kt,
                          (((1,), (1,)), ((), ())),
                          preferred_element_type=jnp.float32)
    qk = lax.dot_general(q, kt, (((1,), (1,)), ((), ())),
                         preferred_element_type=jnp.float32)
    d = (qq + ksq) - 2.0 * qk

    giota = lax.broadcasted_iota(jnp.int32, d.shape, 1) + i * tile

    m1 = jnp.min(d, axis=1, keepdims=True)
    i1 = jnp.min(jnp.where(d == m1, giota, _INT_MAX), axis=1, keepdims=True)
    dm = jnp.where(giota == i1, jnp.inf, d)
    m2 = jnp.min(dm, axis=1, keepdims=True)
    i2 = jnp.min(jnp.where(dm == m2, giota, _INT_MAX), axis=1, keepdims=True)

    v1 = td_ref[:, 0:1]
    v2 = td_ref[:, 1:2]
    j1 = idx_ref[:, 0:1]
    j2 = idx_ref[:, 1:2]

    a = m1 < v1
    b = m2 < v1
    c = m1 < v2
    nv1 = jnp.where(a, m1, v1)
    nj1 = jnp.where(a, i1, j1)
    nv2 = jnp.where(a, jnp.where(b, m2, v1), jnp.where(c, m1, v2))
    nj2 = jnp.where(a, jnp.where(b, i2, j1), jnp.where(c, i1, j2))

    td_ref[...] = jnp.concatenate([nv1, nv2], axis=1)
    idx_ref[...] = jnp.concatenate([nj1, nj2], axis=1)

    onehot = (giota == nj1).astype(jnp.float32)
    r = lax.dot_general(onehot, kt, (((1,), (0,)), ((), ())),
                        preferred_element_type=jnp.float32)
    rt_ref[...] = jnp.where(a, r, rt_ref[...])


@jax.jit
def kernel(queries, keys):
    q_n, dim = queries.shape
    n = keys.shape[0]
    tile = _pick_tile(n)
    grid = (n // tile,)
    out_shape = (
        jax.ShapeDtypeStruct((q_n, 2), jnp.float32),
        jax.ShapeDtypeStruct((q_n, 2), jnp.int32),
        jax.ShapeDtypeStruct((q_n, dim), jnp.float32),
    )
    td, idx, rt = pl.pallas_call(
        functools.partial(_scan_body, tile=tile),
        grid=grid,
        in_specs=[
            pl.BlockSpec((q_n, dim), lambda i: (0, 0)),
            pl.BlockSpec((tile, dim), lambda i: (i, 0)),
        ],
        out_specs=[
            pl.BlockSpec((q_n, 2), lambda i: (0, 0)),
            pl.BlockSpec((q_n, 2), lambda i: (0, 0)),
            pl.BlockSpec((q_n, dim), lambda i: (0, 0)),
        ],
        out_shape=out_shape,
        compiler_params=pltpu.CompilerParams(
            dimension_semantics=("arbitrary",)),
    )(queries, keys)
    return td, idx, rt
